# baseline (speedup 1.0000x reference)
import math

import jax
import jax.numpy as jnp
from jax import lax
from jax.experimental import pallas as pl
from jax.experimental.pallas import tpu as pltpu
from jax.experimental.pallas import tpu_sc as plsc

TABLE_SIZES = ((100000, 3), (39060, 2), (17295, 1), (7424, 2), (20265, 6),
               (3, 1), (7122, 1), (1543, 1), (63, 1), (100000, 7),
               (100000, 3), (100000, 8), (10, 1), (2209, 6), (11938, 9),
               (155, 5), (4, 1), (976, 1), (14, 1), (100000, 12),
               (100000, 100), (100000, 27), (100000, 10), (12973, 3),
               (108, 1), (36, 1))
TABLE_DIMS = tuple(d for _, d in TABLE_SIZES)
NUM_TABLES = len(TABLE_DIMS)
EMB_TOTAL = sum(TABLE_DIMS)
BATCH = 4096
NUM_DENSE = 13
BOTTOM = (512, 256, 128)
TOP = (1024, 1024, 512, 256, 1)
DCN_LAYERS = 3
X0 = BOTTOM[-1] + EMB_TOTAL

TABLE_M = tuple(
    -(-(d + 8 - math.gcd(d, 8)) // 8) for d in TABLE_DIMS)
TABLE_G = tuple(-(-v * d // 8) for (v, d) in TABLE_SIZES)
WIN = tuple(8 * m for m in TABLE_M)

BANKS = ((0, 12), (12, 20), (20, 21), (21, 26))
_woff = []
for _b, (_lo, _hi) in enumerate(BANKS):
    _off = 128 * _b
    for _t in range(_lo, _hi):
        _woff.append(_off)
        _off += WIN[_t]
    assert _off <= 128 * (_b + 1)
WOFF = tuple(_woff)
RTOT = 128 * len(BANKS)

_NC, _NS = 2, 16
_NW = _NC * _NS
_PER = BATCH // _NW

_NBUF = 8
_RAW = tuple(v * d % 8 == 0 for v, d in TABLE_SIZES)
_GOFF = tuple(sum(_PER * m for m in TABLE_M[:t]) for t in range(NUM_TABLES))
_GTOT = sum(_PER * m for m in TABLE_M)
_CHUNKS = tuple((t, c) for t in range(NUM_TABLES) for c in range(TABLE_M[t]))


def _row_pad(v, d):
    u = 8 // math.gcd(8, d)
    return (-v) % u


TABLE_S = tuple(_row_pad(v, d) for v, d in TABLE_SIZES)
TABLE_G2 = tuple((v + s) * d // 8
                 for (v, d), s in zip(TABLE_SIZES, TABLE_S))


def _sc_copy_body(*refs):
    srcs = refs[:NUM_TABLES]
    outs = refs[NUM_TABLES:2 * NUM_TABLES]
    sem = refs[2 * NUM_TABLES]
    wid = lax.axis_index("s") * _NC + lax.axis_index("c")
    cps = []
    rr = 0
    for t in range(NUM_TABLES):
        v, d = TABLE_SIZES[t]
        u = 8 // math.gcd(8, d)
        vp = v + TABLE_S[t]
        if TABLE_S[t] == 0 and v >= 4 * _NW:
            rpt = -(-vp // (_NW * u)) * u
            start = jnp.minimum(wid * rpt, vp - rpt)
            cps.append(pltpu.async_copy(
                srcs[t].at[pl.ds(start, rpt)],
                outs[t].at[pl.ds(start, rpt)], sem))
        else:
            @pl.when(wid == rr % _NW)
            def _(t=t):
                pltpu.async_copy(
                    srcs[t], outs[t].at[pl.ds(0, TABLE_SIZES[t][0])],
                    sem).wait()
            rr += 1
    for cp in cps:
        cp.wait()


def _make_sc_copy():
    mesh = plsc.VectorSubcoreMesh(core_axis_name="c", subcore_axis_name="s",
                                  num_cores=_NC, num_subcores=_NS)
    out_type = [jax.ShapeDtypeStruct((v + s, d), jnp.float32)
                for (v, d), s in zip(TABLE_SIZES, TABLE_S)]
    return pl.kernel(_sc_copy_body, out_type=out_type, mesh=mesh,
                     scratch_types=[pltpu.SemaphoreType.DMA],
                     compiler_params=pltpu.CompilerParams(
                         use_tc_tiling_on_sc=False))


def _sc_gather_body(*refs):
    tables = refs[:NUM_TABLES]
    gidxs = refs[NUM_TABLES:2 * NUM_TABLES]
    outs = refs[2 * NUM_TABLES:3 * NUM_TABLES]
    gbuf = refs[3 * NUM_TABLES]
    bufs = refs[3 * NUM_TABLES + 1:3 * NUM_TABLES + 1 + _NBUF]
    gsems = refs[3 * NUM_TABLES + 1 + _NBUF]
    ssems = refs[3 * NUM_TABLES + 2 + _NBUF]
    isem = refs[3 * NUM_TABLES + 3 + _NBUF]

    wid = lax.axis_index("s") * _NC + lax.axis_index("c")
    base = wid * _PER

    grans = tables

    stage = []
    for t in range(NUM_TABLES):
        m = TABLE_M[t]
        stage.append(pltpu.async_copy(
            gidxs[t].at[pl.ds(base * m, _PER * m)],
            gbuf.at[pl.ds(_GOFF[t], _PER * m)], isem))
    for cp in stage:
        cp.wait()

    n = len(_CHUNKS)

    def fire_gather(k):
        t, c = _CHUNKS[k]
        return pltpu.async_copy(
            grans[t].at[gbuf.at[pl.ds(_GOFF[t] + c * _PER, _PER)]],
            bufs[k % _NBUF], gsems.at[k % _NBUF])

    gcp = [None] * n
    scp = [None] * n
    for k in range(min(_NBUF, n)):
        gcp[k] = fire_gather(k)
    for k in range(n):
        t, c = _CHUNKS[k]
        gcp[k].wait()
        m = TABLE_M[t]
        scp[k] = pltpu.async_copy(
            bufs[k % _NBUF], outs[t].at[pl.ds(base * m + c * _PER, _PER)],
            ssems.at[k % _NBUF])
        nk = k + _NBUF
        if nk < n:
            scp[k].wait()
            gcp[nk] = fire_gather(nk)
    for k in range(max(0, n - _NBUF), n):
        scp[k].wait()


def _make_sc_gather():
    mesh = plsc.VectorSubcoreMesh(core_axis_name="c", subcore_axis_name="s",
                                  num_cores=_NC, num_subcores=_NS)
    out_type = [jax.ShapeDtypeStruct((BATCH * m, 8), jnp.float32)
                for m in TABLE_M]
    scratch = ([pltpu.VMEM((_GTOT,), jnp.int32)] +
               [pltpu.VMEM((_PER, 8), jnp.float32) for _ in range(_NBUF)] +
               [pltpu.SemaphoreType.DMA((_NBUF,)),
                pltpu.SemaphoreType.DMA((_NBUF,)),
                pltpu.SemaphoreType.DMA])
    return pl.kernel(_sc_gather_body, out_type=out_type, mesh=mesh,
                     scratch_types=scratch,
                     compiler_params=pltpu.CompilerParams(
                         use_tc_tiling_on_sc=False))


def _dense_body(dense, rwin, colidx, bw0, bw1, bw2, bb0, bb1, bb2,
                u0, u1, u2, v0, v1, v2, db0, db1, db2,
                tw0, tw1, tw2, tw3, tw4, tb0, tb1, tb2, tb3, tb4, out):
    f32 = jnp.float32
    x = dense[...]
    for bw, bb in ((bw0, bb0), (bw1, bb1), (bw2, bb2)):
        x = jnp.maximum(jnp.dot(x, bw[...], preferred_element_type=f32)
                        + bb[...], 0.0)
    r = rwin[...]
    ci = colidx[...]
    pieces = [x]
    doff = 0
    for b, (lo, hi) in enumerate(BANKS):
        nb = sum(TABLE_DIMS[t] for t in range(lo, hi))
        bank = lax.slice(r, (0, 128 * b), (r.shape[0], 128 * (b + 1)))
        cib = lax.slice(ci, (0, doff), (ci.shape[0], doff + nb))
        pieces.append(jnp.take_along_axis(bank, cib, axis=1))
        doff += nb
    x0 = jnp.concatenate(pieces, axis=1)
    xl = x0
    for u, v, db in ((u0, v0, db0), (u1, v1, db1), (u2, v2, db2)):
        h = jnp.dot(xl, u[...], preferred_element_type=f32)
        h = jnp.dot(h, v[...], preferred_element_type=f32) + db[...]
        xl = x0 * h + xl
    y = xl
    for tw, tb in ((tw0, tb0), (tw1, tb1), (tw2, tb2), (tw3, tb3)):
        y = jnp.maximum(jnp.dot(y, tw[...], preferred_element_type=f32)
                        + tb[...], 0.0)
    y = jnp.dot(y, tw4[...], preferred_element_type=f32) + tb4[...]
    out[...] = y


_TB = 512


def _make_dense():
    grid = (BATCH // _TB,)

    def tile_spec(shape):
        return pl.BlockSpec((_TB,) + shape[1:],
                            lambda i: (i,) + (0,) * (len(shape) - 1))

    def full_spec(shape):
        return pl.BlockSpec(shape, lambda i: (0,) * len(shape))

    in_specs = [
        tile_spec((BATCH, NUM_DENSE)),
        tile_spec((BATCH, RTOT)),
        tile_spec((BATCH, EMB_TOTAL)),
        full_spec((NUM_DENSE, BOTTOM[0])),
        full_spec((BOTTOM[0], BOTTOM[1])),
        full_spec((BOTTOM[1], BOTTOM[2])),
        full_spec((1, BOTTOM[0])),
        full_spec((1, BOTTOM[1])),
        full_spec((1, BOTTOM[2])),
    ]
    for _ in range(DCN_LAYERS):
        in_specs.append(full_spec((X0, 512)))
    for _ in range(DCN_LAYERS):
        in_specs.append(full_spec((512, X0)))
    for _ in range(DCN_LAYERS):
        in_specs.append(full_spec((1, X0)))
    tdims = (X0,) + TOP
    for j in range(len(TOP)):
        in_specs.append(full_spec((tdims[j], tdims[j + 1])))
    for j in range(len(TOP)):
        in_specs.append(full_spec((1, TOP[j])))

    return pl.pallas_call(
        _dense_body,
        grid=grid,
        in_specs=in_specs,
        out_specs=tile_spec((BATCH, 1)),
        out_shape=jax.ShapeDtypeStruct((BATCH, 1), jnp.float32),
    )


def kernel(dense_0, dense_1, dense_2, dense_3, dense_4, dense_5, dense_6,
           dense_7, dense_8, dense_9, dense_10, dense_11, dense_12,
           sparse_idx_0, sparse_idx_1, sparse_idx_2, sparse_idx_3,
           sparse_idx_4, sparse_idx_5, sparse_idx_6, sparse_idx_7,
           sparse_idx_8, sparse_idx_9, sparse_idx_10, sparse_idx_11,
           sparse_idx_12, sparse_idx_13, sparse_idx_14, sparse_idx_15,
           sparse_idx_16, sparse_idx_17, sparse_idx_18, sparse_idx_19,
           sparse_idx_20, sparse_idx_21, sparse_idx_22, sparse_idx_23,
           sparse_idx_24, sparse_idx_25,
           emb_0, emb_1, emb_2, emb_3, emb_4, emb_5, emb_6, emb_7, emb_8,
           emb_9, emb_10, emb_11, emb_12, emb_13, emb_14, emb_15, emb_16,
           emb_17, emb_18, emb_19, emb_20, emb_21, emb_22, emb_23, emb_24,
           emb_25,
           bw_0, bw_1, bw_2, bb_0, bb_1, bb_2,
           u_0, u_1, u_2, v_0, v_1, v_2, dcb_0, dcb_1, dcb_2,
           tw_0, tw_1, tw_2, tw_3, tw_4, tb_0, tb_1, tb_2, tb_3, tb_4):
    dense = jnp.concatenate(
        [dense_0, dense_1, dense_2, dense_3, dense_4, dense_5, dense_6,
         dense_7, dense_8, dense_9, dense_10, dense_11, dense_12], axis=-1)
    tables = (emb_0, emb_1, emb_2, emb_3, emb_4, emb_5, emb_6, emb_7, emb_8,
              emb_9, emb_10, emb_11, emb_12, emb_13, emb_14, emb_15, emb_16,
              emb_17, emb_18, emb_19, emb_20, emb_21, emb_22, emb_23, emb_24,
              emb_25)
    idxs = (sparse_idx_0, sparse_idx_1, sparse_idx_2, sparse_idx_3,
            sparse_idx_4, sparse_idx_5, sparse_idx_6, sparse_idx_7,
            sparse_idx_8, sparse_idx_9, sparse_idx_10, sparse_idx_11,
            sparse_idx_12, sparse_idx_13, sparse_idx_14, sparse_idx_15,
            sparse_idx_16, sparse_idx_17, sparse_idx_18, sparse_idx_19,
            sparse_idx_20, sparse_idx_21, sparse_idx_22, sparse_idx_23,
            sparse_idx_24, sparse_idx_25)

    lin = _make_sc_copy()(*tables)

    flats = []
    gidxs = []
    shifts = []
    for t in range(NUM_TABLES):
        v, d = TABLE_SIZES[t]
        g = TABLE_G[t]
        m = TABLE_M[t]
        flats.append(jnp.reshape(lin[t], (TABLE_G2[t], 8)))
        start = (idxs[t] * d) >> 3
        gi = start[:, None] + jnp.arange(m, dtype=jnp.int32)[None, :]
        gidxs.append(jnp.reshape(jnp.minimum(gi, g - 1), (-1,)))
        shifts.append((idxs[t] * d) & 7)
    cparts = []
    for b, (lo, hi) in enumerate(BANKS):
        for t in range(lo, hi):
            base = WOFF[t] - 128 * b
            cparts.append(
                shifts[t][:, None] +
                jnp.arange(base, base + TABLE_DIMS[t],
                           dtype=jnp.int32)[None, :])
    colidx = jnp.concatenate(cparts, axis=1)

    gathered = _make_sc_gather()(*flats, *gidxs)
    rparts = []
    for b, (lo, hi) in enumerate(BANKS):
        used = 0
        for t in range(lo, hi):
            rparts.append(jnp.reshape(gathered[t], (BATCH, WIN[t])))
            used += WIN[t]
        if used < 128:
            rparts.append(jnp.zeros((BATCH, 128 - used), jnp.float32))
    rwin = jnp.concatenate(rparts, axis=1)

    out = _make_dense()(
        dense, rwin, colidx,
        bw_0, bw_1, bw_2,
        bb_0.reshape(1, -1), bb_1.reshape(1, -1), bb_2.reshape(1, -1),
        u_0, u_1, u_2, v_0, v_1, v_2,
        dcb_0.reshape(1, -1), dcb_1.reshape(1, -1), dcb_2.reshape(1, -1),
        tw_0, tw_1, tw_2, tw_3, tw_4,
        tb_0.reshape(1, -1), tb_1.reshape(1, -1), tb_2.reshape(1, -1),
        tb_3.reshape(1, -1), tb_4.reshape(1, -1))
    return out.reshape(-1)


# speedup vs baseline: 4.7332x; 4.7332x over previous
import math

import jax
import jax.numpy as jnp
from jax import lax
from jax.experimental import pallas as pl
from jax.experimental.pallas import tpu as pltpu
from jax.experimental.pallas import tpu_sc as plsc

TABLE_SIZES = ((100000, 3), (39060, 2), (17295, 1), (7424, 2), (20265, 6),
               (3, 1), (7122, 1), (1543, 1), (63, 1), (100000, 7),
               (100000, 3), (100000, 8), (10, 1), (2209, 6), (11938, 9),
               (155, 5), (4, 1), (976, 1), (14, 1), (100000, 12),
               (100000, 100), (100000, 27), (100000, 10), (12973, 3),
               (108, 1), (36, 1))
TABLE_DIMS = tuple(d for _, d in TABLE_SIZES)
NUM_TABLES = len(TABLE_DIMS)
EMB_TOTAL = sum(TABLE_DIMS)
BATCH = 4096
NUM_DENSE = 13
BOTTOM = (512, 256, 128)
TOP = (1024, 1024, 512, 256, 1)
DCN_LAYERS = 3
X0 = BOTTOM[-1] + EMB_TOTAL

TABLE_M = tuple(
    -(-(d + 8 - math.gcd(d, 8)) // 8) for d in TABLE_DIMS)
TABLE_G = tuple(-(-v * d // 8) for (v, d) in TABLE_SIZES)
WIN = tuple(8 * m for m in TABLE_M)

BANKS = ((0, 12), (12, 20), (20, 21), (21, 26))
_woff = []
for _b, (_lo, _hi) in enumerate(BANKS):
    _off = 128 * _b
    for _t in range(_lo, _hi):
        _woff.append(_off)
        _off += WIN[_t]
    assert _off <= 128 * (_b + 1)
WOFF = tuple(_woff)
RTOT = 128 * len(BANKS)

_NC, _NS = 2, 16
_NW = _NC * _NS
_PER = BATCH // _NW

_NBUF = 8
_RAW = tuple(v * d % 8 == 0 for v, d in TABLE_SIZES)
_GOFF = tuple(sum(_PER * m for m in TABLE_M[:t]) for t in range(NUM_TABLES))
_GTOT = sum(_PER * m for m in TABLE_M)
_CHUNKS = tuple((t, c) for t in range(NUM_TABLES) for c in range(TABLE_M[t]))


def _row_pad(v, d):
    u = 8 // math.gcd(8, d)
    return (-v) % u


TABLE_S = tuple(_row_pad(v, d) for v, d in TABLE_SIZES)
TABLE_G2 = tuple((v + s) * d // 8
                 for (v, d), s in zip(TABLE_SIZES, TABLE_S))


def _sc_copy_body(*refs):
    srcs = refs[:NUM_TABLES]
    outs = refs[NUM_TABLES:2 * NUM_TABLES]
    sem = refs[2 * NUM_TABLES]
    wid = lax.axis_index("s") * _NC + lax.axis_index("c")
    cps = []
    rr = 0
    for t in range(NUM_TABLES):
        v, d = TABLE_SIZES[t]
        u = 8 // math.gcd(8, d)
        vp = v + TABLE_S[t]
        if TABLE_S[t] == 0 and v >= 4 * _NW:
            rpt = -(-vp // (_NW * u)) * u
            start = jnp.minimum(wid * rpt, vp - rpt)
            cps.append(pltpu.async_copy(
                srcs[t].at[pl.ds(start, rpt)],
                outs[t].at[pl.ds(start, rpt)], sem))
        else:
            @pl.when(wid == rr % _NW)
            def _(t=t):
                pltpu.async_copy(
                    srcs[t], outs[t].at[pl.ds(0, TABLE_SIZES[t][0])],
                    sem).wait()
            rr += 1
    for cp in cps:
        cp.wait()


def _make_sc_copy():
    mesh = plsc.VectorSubcoreMesh(core_axis_name="c", subcore_axis_name="s",
                                  num_cores=_NC, num_subcores=_NS)
    out_type = [jax.ShapeDtypeStruct((v + s, d), jnp.float32)
                for (v, d), s in zip(TABLE_SIZES, TABLE_S)]
    return pl.kernel(_sc_copy_body, out_type=out_type, mesh=mesh,
                     scratch_types=[pltpu.SemaphoreType.DMA],
                     compiler_params=pltpu.CompilerParams(
                         use_tc_tiling_on_sc=False))


def _sc_gather_body(*refs):
    tables = refs[:NUM_TABLES]
    gidxs = refs[NUM_TABLES:2 * NUM_TABLES]
    outs = refs[2 * NUM_TABLES:3 * NUM_TABLES]
    gbuf = refs[3 * NUM_TABLES]
    bufs = refs[3 * NUM_TABLES + 1:3 * NUM_TABLES + 1 + _NBUF]
    gsems = refs[3 * NUM_TABLES + 1 + _NBUF]
    ssems = refs[3 * NUM_TABLES + 2 + _NBUF]
    isem = refs[3 * NUM_TABLES + 3 + _NBUF]

    wid = lax.axis_index("s") * _NC + lax.axis_index("c")
    base = wid * _PER

    grans = tables

    stage = []
    for t in range(NUM_TABLES):
        m = TABLE_M[t]
        stage.append(pltpu.async_copy(
            gidxs[t].at[pl.ds(base * m, _PER * m)],
            gbuf.at[pl.ds(_GOFF[t], _PER * m)], isem))
    for cp in stage:
        cp.wait()

    n = len(_CHUNKS)

    def fire_gather(k):
        t, c = _CHUNKS[k]
        return pltpu.async_copy(
            grans[t].at[gbuf.at[pl.ds(_GOFF[t] + c * _PER, _PER)]],
            bufs[k % _NBUF], gsems.at[k % _NBUF])

    gcp = [None] * n
    scp = [None] * n
    for k in range(min(_NBUF, n)):
        gcp[k] = fire_gather(k)
    for k in range(n):
        t, c = _CHUNKS[k]
        gcp[k].wait()
        m = TABLE_M[t]
        scp[k] = pltpu.async_copy(
            bufs[k % _NBUF], outs[t].at[pl.ds(base * m + c * _PER, _PER)],
            ssems.at[k % _NBUF])
        nk = k + _NBUF
        if nk < n:
            scp[k].wait()
            gcp[nk] = fire_gather(nk)
    for k in range(max(0, n - _NBUF), n):
        scp[k].wait()


def _make_sc_gather():
    mesh = plsc.VectorSubcoreMesh(core_axis_name="c", subcore_axis_name="s",
                                  num_cores=_NC, num_subcores=_NS)
    out_type = [jax.ShapeDtypeStruct((BATCH * m, 8), jnp.float32)
                for m in TABLE_M]
    scratch = ([pltpu.VMEM((_GTOT,), jnp.int32)] +
               [pltpu.VMEM((_PER, 8), jnp.float32) for _ in range(_NBUF)] +
               [pltpu.SemaphoreType.DMA((_NBUF,)),
                pltpu.SemaphoreType.DMA((_NBUF,)),
                pltpu.SemaphoreType.DMA])
    return pl.kernel(_sc_gather_body, out_type=out_type, mesh=mesh,
                     scratch_types=scratch,
                     compiler_params=pltpu.CompilerParams(
                         use_tc_tiling_on_sc=False))


def _dense_body(dense, rwin, colidx, bw0, bw1, bw2, bb0, bb1, bb2,
                u0, u1, u2, v0, v1, v2, db0, db1, db2,
                tw0, tw1, tw2, tw3, tw4, tb0, tb1, tb2, tb3, tb4, out):
    f32 = jnp.float32
    x = dense[...]
    for bw, bb in ((bw0, bb0), (bw1, bb1), (bw2, bb2)):
        x = jnp.maximum(jnp.dot(x, bw[...], preferred_element_type=f32)
                        + bb[...], 0.0)
    r = rwin[...]
    ci = colidx[...]
    pieces = [x]
    doff = 0
    for b, (lo, hi) in enumerate(BANKS):
        nb = sum(TABLE_DIMS[t] for t in range(lo, hi))
        bank = lax.slice(r, (0, 128 * b), (r.shape[0], 128 * (b + 1)))
        cib = lax.slice(ci, (0, doff), (ci.shape[0], doff + nb))
        pieces.append(jnp.take_along_axis(bank, cib, axis=1))
        doff += nb
    x0 = jnp.concatenate(pieces, axis=1)
    xl = x0
    for u, v, db in ((u0, v0, db0), (u1, v1, db1), (u2, v2, db2)):
        h = jnp.dot(xl, u[...], preferred_element_type=f32)
        h = jnp.dot(h, v[...], preferred_element_type=f32) + db[...]
        xl = x0 * h + xl
    y = xl
    for tw, tb in ((tw0, tb0), (tw1, tb1), (tw2, tb2), (tw3, tb3)):
        y = jnp.maximum(jnp.dot(y, tw[...], preferred_element_type=f32)
                        + tb[...], 0.0)
    y = jnp.dot(y, tw4[...], preferred_element_type=f32) + tb4[...]
    out[...] = y


_TB = 512


def _make_dense():
    grid = (BATCH // _TB,)

    def tile_spec(shape):
        return pl.BlockSpec((_TB,) + shape[1:],
                            lambda i: (i,) + (0,) * (len(shape) - 1))

    def full_spec(shape):
        return pl.BlockSpec(shape, lambda i: (0,) * len(shape))

    in_specs = [
        tile_spec((BATCH, NUM_DENSE)),
        tile_spec((BATCH, RTOT)),
        tile_spec((BATCH, EMB_TOTAL)),
        full_spec((NUM_DENSE, BOTTOM[0])),
        full_spec((BOTTOM[0], BOTTOM[1])),
        full_spec((BOTTOM[1], BOTTOM[2])),
        full_spec((1, BOTTOM[0])),
        full_spec((1, BOTTOM[1])),
        full_spec((1, BOTTOM[2])),
    ]
    for _ in range(DCN_LAYERS):
        in_specs.append(full_spec((X0, 512)))
    for _ in range(DCN_LAYERS):
        in_specs.append(full_spec((512, X0)))
    for _ in range(DCN_LAYERS):
        in_specs.append(full_spec((1, X0)))
    tdims = (X0,) + TOP
    for j in range(len(TOP)):
        in_specs.append(full_spec((tdims[j], tdims[j + 1])))
    for j in range(len(TOP)):
        in_specs.append(full_spec((1, TOP[j])))

    return pl.pallas_call(
        _dense_body,
        grid=grid,
        in_specs=in_specs,
        out_specs=tile_spec((BATCH, 1)),
        out_shape=jax.ShapeDtypeStruct((BATCH, 1), jnp.float32),
    )


def kernel(dense_0, dense_1, dense_2, dense_3, dense_4, dense_5, dense_6,
           dense_7, dense_8, dense_9, dense_10, dense_11, dense_12,
           sparse_idx_0, sparse_idx_1, sparse_idx_2, sparse_idx_3,
           sparse_idx_4, sparse_idx_5, sparse_idx_6, sparse_idx_7,
           sparse_idx_8, sparse_idx_9, sparse_idx_10, sparse_idx_11,
           sparse_idx_12, sparse_idx_13, sparse_idx_14, sparse_idx_15,
           sparse_idx_16, sparse_idx_17, sparse_idx_18, sparse_idx_19,
           sparse_idx_20, sparse_idx_21, sparse_idx_22, sparse_idx_23,
           sparse_idx_24, sparse_idx_25,
           emb_0, emb_1, emb_2, emb_3, emb_4, emb_5, emb_6, emb_7, emb_8,
           emb_9, emb_10, emb_11, emb_12, emb_13, emb_14, emb_15, emb_16,
           emb_17, emb_18, emb_19, emb_20, emb_21, emb_22, emb_23, emb_24,
           emb_25,
           bw_0, bw_1, bw_2, bb_0, bb_1, bb_2,
           u_0, u_1, u_2, v_0, v_1, v_2, dcb_0, dcb_1, dcb_2,
           tw_0, tw_1, tw_2, tw_3, tw_4, tb_0, tb_1, tb_2, tb_3, tb_4):
    dense = jnp.concatenate(
        [dense_0, dense_1, dense_2, dense_3, dense_4, dense_5, dense_6,
         dense_7, dense_8, dense_9, dense_10, dense_11, dense_12], axis=-1)
    tables = (emb_0, emb_1, emb_2, emb_3, emb_4, emb_5, emb_6, emb_7, emb_8,
              emb_9, emb_10, emb_11, emb_12, emb_13, emb_14, emb_15, emb_16,
              emb_17, emb_18, emb_19, emb_20, emb_21, emb_22, emb_23, emb_24,
              emb_25)
    idxs = (sparse_idx_0, sparse_idx_1, sparse_idx_2, sparse_idx_3,
            sparse_idx_4, sparse_idx_5, sparse_idx_6, sparse_idx_7,
            sparse_idx_8, sparse_idx_9, sparse_idx_10, sparse_idx_11,
            sparse_idx_12, sparse_idx_13, sparse_idx_14, sparse_idx_15,
            sparse_idx_16, sparse_idx_17, sparse_idx_18, sparse_idx_19,
            sparse_idx_20, sparse_idx_21, sparse_idx_22, sparse_idx_23,
            sparse_idx_24, sparse_idx_25)

    flats = []
    gidxs = []
    shifts = []
    for t in range(NUM_TABLES):
        v, d = TABLE_SIZES[t]
        g = TABLE_G[t]
        m = TABLE_M[t]
        flat = jnp.reshape(tables[t], (-1,))
        if v * d != g * 8:
            flat = jnp.pad(flat, (0, g * 8 - v * d))
        flats.append(jnp.reshape(flat, (g, 8)))
        start = (idxs[t] * d) >> 3
        gi = start[:, None] + jnp.arange(m, dtype=jnp.int32)[None, :]
        gidxs.append(jnp.reshape(jnp.minimum(gi, g - 1), (-1,)))
        shifts.append((idxs[t] * d) & 7)
    cparts = []
    for b, (lo, hi) in enumerate(BANKS):
        for t in range(lo, hi):
            base = WOFF[t] - 128 * b
            cparts.append(
                shifts[t][:, None] +
                jnp.arange(base, base + TABLE_DIMS[t],
                           dtype=jnp.int32)[None, :])
    colidx = jnp.concatenate(cparts, axis=1)

    gathered = _make_sc_gather()(*flats, *gidxs)
    rparts = []
    for b, (lo, hi) in enumerate(BANKS):
        used = 0
        for t in range(lo, hi):
            rparts.append(jnp.reshape(gathered[t], (BATCH, WIN[t])))
            used += WIN[t]
        if used < 128:
            rparts.append(jnp.zeros((BATCH, 128 - used), jnp.float32))
    rwin = jnp.concatenate(rparts, axis=1)

    out = _make_dense()(
        dense, rwin, colidx,
        bw_0, bw_1, bw_2,
        bb_0.reshape(1, -1), bb_1.reshape(1, -1), bb_2.reshape(1, -1),
        u_0, u_1, u_2, v_0, v_1, v_2,
        dcb_0.reshape(1, -1), dcb_1.reshape(1, -1), dcb_2.reshape(1, -1),
        tw_0, tw_1, tw_2, tw_3, tw_4,
        tb_0.reshape(1, -1), tb_1.reshape(1, -1), tb_2.reshape(1, -1),
        tb_3.reshape(1, -1), tb_4.reshape(1, -1))
    return out.reshape(-1)


# speedup vs baseline: 4.8631x; 1.0275x over previous
import math

import jax
import jax.experimental.layout as jlayout
import jax.numpy as jnp
from jax import lax
from jax.experimental import pallas as pl
from jax.experimental.pallas import tpu as pltpu
from jax.experimental.pallas import tpu_sc as plsc

TABLE_SIZES = ((100000, 3), (39060, 2), (17295, 1), (7424, 2), (20265, 6),
               (3, 1), (7122, 1), (1543, 1), (63, 1), (100000, 7),
               (100000, 3), (100000, 8), (10, 1), (2209, 6), (11938, 9),
               (155, 5), (4, 1), (976, 1), (14, 1), (100000, 12),
               (100000, 100), (100000, 27), (100000, 10), (12973, 3),
               (108, 1), (36, 1))
TABLE_DIMS = tuple(d for _, d in TABLE_SIZES)
NUM_TABLES = len(TABLE_DIMS)
EMB_TOTAL = sum(TABLE_DIMS)
BATCH = 4096
NUM_DENSE = 13
BOTTOM = (512, 256, 128)
TOP = (1024, 1024, 512, 256, 1)
DCN_LAYERS = 3
X0 = BOTTOM[-1] + EMB_TOTAL

TABLE_M = tuple(
    -(-(d + 8 - math.gcd(d, 8)) // 8) for d in TABLE_DIMS)
TABLE_G = tuple(-(-v * d // 8) for (v, d) in TABLE_SIZES)
WIN = tuple(8 * m for m in TABLE_M)

BANKS = ((0, 12), (12, 20), (20, 21), (21, 26))
_woff = []
for _b, (_lo, _hi) in enumerate(BANKS):
    _off = 128 * _b
    for _t in range(_lo, _hi):
        _woff.append(_off)
        _off += WIN[_t]
    assert _off <= 128 * (_b + 1)
WOFF = tuple(_woff)
RTOT = 128 * len(BANKS)

_NC, _NS = 2, 16
_NW = _NC * _NS
_PER = BATCH // _NW

_NBUF = 8
_RAW = tuple(v * d % 8 == 0 for v, d in TABLE_SIZES)
_GOFF = tuple(sum(_PER * m for m in TABLE_M[:t]) for t in range(NUM_TABLES))
_GTOT = sum(_PER * m for m in TABLE_M)
_CHUNKS = tuple((t, c) for t in range(NUM_TABLES) for c in range(TABLE_M[t]))


def _row_pad(v, d):
    u = 8 // math.gcd(8, d)
    return (-v) % u


TABLE_S = tuple(_row_pad(v, d) for v, d in TABLE_SIZES)
TABLE_G2 = tuple((v + s) * d // 8
                 for (v, d), s in zip(TABLE_SIZES, TABLE_S))


def _sc_copy_body(*refs):
    srcs = refs[:NUM_TABLES]
    outs = refs[NUM_TABLES:2 * NUM_TABLES]
    sem = refs[2 * NUM_TABLES]
    wid = lax.axis_index("s") * _NC + lax.axis_index("c")
    cps = []
    rr = 0
    for t in range(NUM_TABLES):
        v, d = TABLE_SIZES[t]
        u = 8 // math.gcd(8, d)
        vp = v + TABLE_S[t]
        if TABLE_S[t] == 0 and v >= 4 * _NW:
            rpt = -(-vp // (_NW * u)) * u
            start = jnp.minimum(wid * rpt, vp - rpt)
            cps.append(pltpu.async_copy(
                srcs[t].at[pl.ds(start, rpt)],
                outs[t].at[pl.ds(start, rpt)], sem))
        else:
            @pl.when(wid == rr % _NW)
            def _(t=t):
                pltpu.async_copy(
                    srcs[t], outs[t].at[pl.ds(0, TABLE_SIZES[t][0])],
                    sem).wait()
            rr += 1
    for cp in cps:
        cp.wait()


def _make_sc_copy():
    mesh = plsc.VectorSubcoreMesh(core_axis_name="c", subcore_axis_name="s",
                                  num_cores=_NC, num_subcores=_NS)
    out_type = [jax.ShapeDtypeStruct((v + s, d), jnp.float32)
                for (v, d), s in zip(TABLE_SIZES, TABLE_S)]
    return pl.kernel(_sc_copy_body, out_type=out_type, mesh=mesh,
                     scratch_types=[pltpu.SemaphoreType.DMA],
                     compiler_params=pltpu.CompilerParams(
                         use_tc_tiling_on_sc=False))


def _sc_gather_body(*refs):
    tables = refs[:NUM_TABLES]
    gidxs = refs[NUM_TABLES:2 * NUM_TABLES]
    outs = refs[2 * NUM_TABLES:3 * NUM_TABLES]
    gbuf = refs[3 * NUM_TABLES]
    bufs = refs[3 * NUM_TABLES + 1:3 * NUM_TABLES + 1 + _NBUF]
    gsems = refs[3 * NUM_TABLES + 1 + _NBUF]
    ssems = refs[3 * NUM_TABLES + 2 + _NBUF]
    isem = refs[3 * NUM_TABLES + 3 + _NBUF]

    wid = lax.axis_index("s") * _NC + lax.axis_index("c")
    base = wid * _PER

    grans = tables

    stage = []
    for t in range(NUM_TABLES):
        m = TABLE_M[t]
        stage.append(pltpu.async_copy(
            gidxs[t].at[pl.ds(base * m, _PER * m)],
            gbuf.at[pl.ds(_GOFF[t], _PER * m)], isem))
    for cp in stage:
        cp.wait()

    n = len(_CHUNKS)

    def fire_gather(k):
        t, c = _CHUNKS[k]
        return pltpu.async_copy(
            grans[t].at[gbuf.at[pl.ds(_GOFF[t] + c * _PER, _PER)]],
            bufs[k % _NBUF], gsems.at[k % _NBUF])

    gcp = [None] * n
    scp = [None] * n
    for k in range(min(_NBUF, n)):
        gcp[k] = fire_gather(k)
    for k in range(n):
        t, c = _CHUNKS[k]
        gcp[k].wait()
        m = TABLE_M[t]
        scp[k] = pltpu.async_copy(
            bufs[k % _NBUF], outs[t].at[pl.ds(base * m + c * _PER, _PER)],
            ssems.at[k % _NBUF])
        nk = k + _NBUF
        if nk < n:
            scp[k].wait()
            gcp[nk] = fire_gather(nk)
    for k in range(max(0, n - _NBUF), n):
        scp[k].wait()


def _make_sc_gather():
    mesh = plsc.VectorSubcoreMesh(core_axis_name="c", subcore_axis_name="s",
                                  num_cores=_NC, num_subcores=_NS)
    out_type = [jax.ShapeDtypeStruct((BATCH * m, 8), jnp.float32)
                for m in TABLE_M]
    scratch = ([pltpu.VMEM((_GTOT,), jnp.int32)] +
               [pltpu.VMEM((_PER, 8), jnp.float32) for _ in range(_NBUF)] +
               [pltpu.SemaphoreType.DMA((_NBUF,)),
                pltpu.SemaphoreType.DMA((_NBUF,)),
                pltpu.SemaphoreType.DMA])
    return pl.kernel(_sc_gather_body, out_type=out_type, mesh=mesh,
                     scratch_types=scratch,
                     compiler_params=pltpu.CompilerParams(
                         use_tc_tiling_on_sc=False))


def _dense_body(dense, rwin, colidx, bw0, bw1, bw2, bb0, bb1, bb2,
                u0, u1, u2, v0, v1, v2, db0, db1, db2,
                tw0, tw1, tw2, tw3, tw4, tb0, tb1, tb2, tb3, tb4, out):
    f32 = jnp.float32
    x = dense[...]
    for bw, bb in ((bw0, bb0), (bw1, bb1), (bw2, bb2)):
        x = jnp.maximum(jnp.dot(x, bw[...], preferred_element_type=f32)
                        + bb[...], 0.0)
    r = rwin[...]
    ci = colidx[...]
    pieces = [x]
    doff = 0
    for b, (lo, hi) in enumerate(BANKS):
        nb = sum(TABLE_DIMS[t] for t in range(lo, hi))
        bank = lax.slice(r, (0, 128 * b), (r.shape[0], 128 * (b + 1)))
        cib = lax.slice(ci, (0, doff), (ci.shape[0], doff + nb))
        pieces.append(jnp.take_along_axis(bank, cib, axis=1))
        doff += nb
    x0 = jnp.concatenate(pieces, axis=1)
    xl = x0
    for u, v, db in ((u0, v0, db0), (u1, v1, db1), (u2, v2, db2)):
        h = jnp.dot(xl, u[...], preferred_element_type=f32)
        h = jnp.dot(h, v[...], preferred_element_type=f32) + db[...]
        xl = x0 * h + xl
    y = xl
    for tw, tb in ((tw0, tb0), (tw1, tb1), (tw2, tb2), (tw3, tb3)):
        y = jnp.maximum(jnp.dot(y, tw[...], preferred_element_type=f32)
                        + tb[...], 0.0)
    y = jnp.dot(y, tw4[...], preferred_element_type=f32) + tb4[...]
    out[...] = y


_TB = 512


def _make_dense():
    grid = (BATCH // _TB,)

    def tile_spec(shape):
        return pl.BlockSpec((_TB,) + shape[1:],
                            lambda i: (i,) + (0,) * (len(shape) - 1))

    def full_spec(shape):
        return pl.BlockSpec(shape, lambda i: (0,) * len(shape))

    in_specs = [
        tile_spec((BATCH, NUM_DENSE)),
        tile_spec((BATCH, RTOT)),
        tile_spec((BATCH, EMB_TOTAL)),
        full_spec((NUM_DENSE, BOTTOM[0])),
        full_spec((BOTTOM[0], BOTTOM[1])),
        full_spec((BOTTOM[1], BOTTOM[2])),
        full_spec((1, BOTTOM[0])),
        full_spec((1, BOTTOM[1])),
        full_spec((1, BOTTOM[2])),
    ]
    for _ in range(DCN_LAYERS):
        in_specs.append(full_spec((X0, 512)))
    for _ in range(DCN_LAYERS):
        in_specs.append(full_spec((512, X0)))
    for _ in range(DCN_LAYERS):
        in_specs.append(full_spec((1, X0)))
    tdims = (X0,) + TOP
    for j in range(len(TOP)):
        in_specs.append(full_spec((tdims[j], tdims[j + 1])))
    for j in range(len(TOP)):
        in_specs.append(full_spec((1, TOP[j])))

    return pl.pallas_call(
        _dense_body,
        grid=grid,
        in_specs=in_specs,
        out_specs=tile_spec((BATCH, 1)),
        out_shape=jax.ShapeDtypeStruct((BATCH, 1), jnp.float32),
    )


def kernel(dense_0, dense_1, dense_2, dense_3, dense_4, dense_5, dense_6,
           dense_7, dense_8, dense_9, dense_10, dense_11, dense_12,
           sparse_idx_0, sparse_idx_1, sparse_idx_2, sparse_idx_3,
           sparse_idx_4, sparse_idx_5, sparse_idx_6, sparse_idx_7,
           sparse_idx_8, sparse_idx_9, sparse_idx_10, sparse_idx_11,
           sparse_idx_12, sparse_idx_13, sparse_idx_14, sparse_idx_15,
           sparse_idx_16, sparse_idx_17, sparse_idx_18, sparse_idx_19,
           sparse_idx_20, sparse_idx_21, sparse_idx_22, sparse_idx_23,
           sparse_idx_24, sparse_idx_25,
           emb_0, emb_1, emb_2, emb_3, emb_4, emb_5, emb_6, emb_7, emb_8,
           emb_9, emb_10, emb_11, emb_12, emb_13, emb_14, emb_15, emb_16,
           emb_17, emb_18, emb_19, emb_20, emb_21, emb_22, emb_23, emb_24,
           emb_25,
           bw_0, bw_1, bw_2, bb_0, bb_1, bb_2,
           u_0, u_1, u_2, v_0, v_1, v_2, dcb_0, dcb_1, dcb_2,
           tw_0, tw_1, tw_2, tw_3, tw_4, tb_0, tb_1, tb_2, tb_3, tb_4):
    dense = jnp.concatenate(
        [dense_0, dense_1, dense_2, dense_3, dense_4, dense_5, dense_6,
         dense_7, dense_8, dense_9, dense_10, dense_11, dense_12], axis=-1)
    tables = (emb_0, emb_1, emb_2, emb_3, emb_4, emb_5, emb_6, emb_7, emb_8,
              emb_9, emb_10, emb_11, emb_12, emb_13, emb_14, emb_15, emb_16,
              emb_17, emb_18, emb_19, emb_20, emb_21, emb_22, emb_23, emb_24,
              emb_25)
    idxs = (sparse_idx_0, sparse_idx_1, sparse_idx_2, sparse_idx_3,
            sparse_idx_4, sparse_idx_5, sparse_idx_6, sparse_idx_7,
            sparse_idx_8, sparse_idx_9, sparse_idx_10, sparse_idx_11,
            sparse_idx_12, sparse_idx_13, sparse_idx_14, sparse_idx_15,
            sparse_idx_16, sparse_idx_17, sparse_idx_18, sparse_idx_19,
            sparse_idx_20, sparse_idx_21, sparse_idx_22, sparse_idx_23,
            sparse_idx_24, sparse_idx_25)

    flats = []
    gidxs = []
    shifts = []
    for t in range(NUM_TABLES):
        v, d = TABLE_SIZES[t]
        g = TABLE_G[t]
        m = TABLE_M[t]
        trm = jlayout.with_layout_constraint(tables[t], jlayout.Layout((0, 1)))
        flat = jnp.reshape(trm, (-1,))
        if v * d != g * 8:
            flat = jnp.pad(flat, (0, g * 8 - v * d))
        flats.append(jnp.reshape(flat, (g, 8)))
        start = (idxs[t] * d) >> 3
        gi = start[:, None] + jnp.arange(m, dtype=jnp.int32)[None, :]
        gidxs.append(jnp.reshape(jnp.minimum(gi, g - 1), (-1,)))
        shifts.append((idxs[t] * d) & 7)
    cparts = []
    for b, (lo, hi) in enumerate(BANKS):
        for t in range(lo, hi):
            base = WOFF[t] - 128 * b
            cparts.append(
                shifts[t][:, None] +
                jnp.arange(base, base + TABLE_DIMS[t],
                           dtype=jnp.int32)[None, :])
    colidx = jnp.concatenate(cparts, axis=1)

    gathered = _make_sc_gather()(*flats, *gidxs)
    rparts = []
    for b, (lo, hi) in enumerate(BANKS):
        used = 0
        for t in range(lo, hi):
            rparts.append(jnp.reshape(gathered[t], (BATCH, WIN[t])))
            used += WIN[t]
        if used < 128:
            rparts.append(jnp.zeros((BATCH, 128 - used), jnp.float32))
    rwin = jnp.concatenate(rparts, axis=1)

    out = _make_dense()(
        dense, rwin, colidx,
        bw_0, bw_1, bw_2,
        bb_0.reshape(1, -1), bb_1.reshape(1, -1), bb_2.reshape(1, -1),
        u_0, u_1, u_2, v_0, v_1, v_2,
        dcb_0.reshape(1, -1), dcb_1.reshape(1, -1), dcb_2.reshape(1, -1),
        tw_0, tw_1, tw_2, tw_3, tw_4,
        tb_0.reshape(1, -1), tb_1.reshape(1, -1), tb_2.reshape(1, -1),
        tb_3.reshape(1, -1), tb_4.reshape(1, -1))
    return out.reshape(-1)
